# initial kernel scaffold (unmeasured)
import functools

import jax
import jax.numpy as jnp
from jax import lax
from jax.experimental import pallas as pl
from jax.experimental.pallas import tpu as pltpu

N_DEV = 16
ROWS = 4096
D = 1024


def _allgather_body(x_ref, dest_ref, xg_ref, dg_ref,
                    copy_sem, x_send, x_recv, d_send, d_recv):
    me = lax.axis_index("i")
    left = (me - 1) % N_DEV
    right = (me + 1) % N_DEV

    barrier = pltpu.get_barrier_semaphore()
    for nbr in (left, right):
        pl.semaphore_signal(barrier, inc=1, device_id=(nbr,),
                            device_id_type=pl.DeviceIdType.MESH)
    pl.semaphore_wait(barrier, 2)

    cp = pltpu.make_async_copy(x_ref, xg_ref.at[me], copy_sem)
    cp.start()
    dg_ref[pl.ds(me, 1), :] = dest_ref[...]
    cp.wait()

    for h in range(N_DEV - 1):
        slot = (me - h) % N_DEV
        rx = pltpu.make_async_remote_copy(
            src_ref=xg_ref.at[slot],
            dst_ref=xg_ref.at[slot],
            send_sem=x_send.at[h],
            recv_sem=x_recv.at[h],
            device_id=(right,),
            device_id_type=pl.DeviceIdType.MESH,
        )
        rd = pltpu.make_async_remote_copy(
            src_ref=dg_ref.at[pl.ds(slot, 1)],
            dst_ref=dg_ref.at[pl.ds(slot, 1)],
            send_sem=d_send.at[h],
            recv_sem=d_recv.at[h],
            device_id=(right,),
            device_id_type=pl.DeviceIdType.MESH,
        )
        rx.start()
        rd.start()
        rx.wait()
        rd.wait()

    @functools.partial(pl.run_scoped, exit_sem=pltpu.SemaphoreType.REGULAR)
    def _(exit_sem):
        for nbr in (left, right):
            pl.semaphore_signal(exit_sem, inc=1, device_id=(nbr,),
                                device_id_type=pl.DeviceIdType.MESH)
        pl.semaphore_wait(exit_sem, 2)


def kernel(x, dest):
    xb = x.astype(jnp.bfloat16)
    d2 = dest.astype(jnp.int32).reshape(1, ROWS)

    xg, dg = pl.pallas_call(
        _allgather_body,
        out_shape=(
            jax.ShapeDtypeStruct((N_DEV, ROWS, D), jnp.bfloat16),
            jax.ShapeDtypeStruct((N_DEV, ROWS), jnp.int32),
        ),
        in_specs=[
            pl.BlockSpec(memory_space=pltpu.VMEM),
            pl.BlockSpec(memory_space=pltpu.VMEM),
        ],
        out_specs=(
            pl.BlockSpec(memory_space=pltpu.ANY),
            pl.BlockSpec(memory_space=pltpu.VMEM),
        ),
        scratch_shapes=[
            pltpu.SemaphoreType.DMA,
            pltpu.SemaphoreType.DMA((N_DEV - 1,)),
            pltpu.SemaphoreType.DMA((N_DEV - 1,)),
            pltpu.SemaphoreType.DMA((N_DEV - 1,)),
            pltpu.SemaphoreType.DMA((N_DEV - 1,)),
        ],
        compiler_params=pltpu.CompilerParams(collective_id=0),
    )(xb, d2)

    me = lax.axis_index("i")
    order = jnp.argsort(dg.reshape(-1), stable=True)
    mine = lax.dynamic_slice_in_dim(order, me * ROWS, ROWS)
    return jnp.take(xg.reshape(N_DEV * ROWS, D), mine, axis=0)


# baseline (device time: 1659735 ns/iter reference)
import functools

import jax
import jax.numpy as jnp
from jax import lax
from jax.experimental import pallas as pl
from jax.experimental.pallas import tpu as pltpu

N_DEV = 16
ROWS = 4096
D = 1024


def _allgather_body(x_ref, dest_ref, xg_ref, dg_ref,
                    copy_sem, x_send, x_recv, d_send, d_recv):
    me = lax.axis_index("i")
    left = (me - 1) % N_DEV
    right = (me + 1) % N_DEV

    barrier = pltpu.get_barrier_semaphore()
    for nbr in (left, right):
        pl.semaphore_signal(barrier, inc=1, device_id=(nbr,),
                            device_id_type=pl.DeviceIdType.MESH)
    pl.semaphore_wait(barrier, 2)

    cp = pltpu.make_async_copy(x_ref, xg_ref.at[me], copy_sem)
    cp.start()
    dg_ref[pl.ds(me, 1), :] = dest_ref[...]
    cp.wait()

    for h in range(N_DEV - 1):
        slot = (me - h) % N_DEV
        rx = pltpu.make_async_remote_copy(
            src_ref=xg_ref.at[slot],
            dst_ref=xg_ref.at[slot],
            send_sem=x_send.at[h],
            recv_sem=x_recv.at[h],
            device_id=(right,),
            device_id_type=pl.DeviceIdType.MESH,
        )
        rd = pltpu.make_async_remote_copy(
            src_ref=dg_ref.at[pl.ds(slot, 1)],
            dst_ref=dg_ref.at[pl.ds(slot, 1)],
            send_sem=d_send.at[h],
            recv_sem=d_recv.at[h],
            device_id=(right,),
            device_id_type=pl.DeviceIdType.MESH,
        )
        rx.start()
        rd.start()
        rx.wait()
        rd.wait()

    @functools.partial(pl.run_scoped, exit_sem=pltpu.SemaphoreType.REGULAR)
    def _(exit_sem):
        for nbr in (left, right):
            pl.semaphore_signal(exit_sem, inc=1, device_id=(nbr,),
                                device_id_type=pl.DeviceIdType.MESH)
        pl.semaphore_wait(exit_sem, 2)


def kernel(x, dest):
    xb = x.astype(jnp.bfloat16)
    d2 = dest.astype(jnp.int32).reshape(1, ROWS)

    xg, dg = pl.pallas_call(
        _allgather_body,
        out_shape=(
            jax.ShapeDtypeStruct((N_DEV, ROWS, D), jnp.bfloat16),
            jax.ShapeDtypeStruct((N_DEV, ROWS), jnp.int32),
        ),
        in_specs=[
            pl.BlockSpec(memory_space=pltpu.VMEM),
            pl.BlockSpec(memory_space=pltpu.VMEM),
        ],
        out_specs=(
            pl.BlockSpec(memory_space=pl.ANY),
            pl.BlockSpec(memory_space=pltpu.VMEM),
        ),
        scratch_shapes=[
            pltpu.SemaphoreType.DMA,
            pltpu.SemaphoreType.DMA((N_DEV - 1,)),
            pltpu.SemaphoreType.DMA((N_DEV - 1,)),
            pltpu.SemaphoreType.DMA((N_DEV - 1,)),
            pltpu.SemaphoreType.DMA((N_DEV - 1,)),
        ],
        compiler_params=pltpu.CompilerParams(collective_id=0),
    )(xb, d2)

    me = lax.axis_index("i")
    order = jnp.argsort(dg.reshape(-1), stable=True)
    mine = lax.dynamic_slice_in_dim(order, me * ROWS, ROWS)
    return jnp.take(xg.reshape(N_DEV * ROWS, D), mine, axis=0)


# device time: 156705 ns/iter; 10.5915x vs baseline; 10.5915x over previous
import jax
import jax.numpy as jnp
from jax import lax
from jax.experimental import pallas as pl
from jax.experimental.pallas import tpu as pltpu

N_DEV = 16
ROWS = 4096
D = 1024
NBITS = 13


def _counts_body(cnt_ref, c_ref, send_sems, recv_sems):
    me = lax.axis_index("i")

    barrier = pltpu.get_barrier_semaphore()
    for k in range(1, N_DEV):
        pl.semaphore_signal(barrier, inc=1, device_id=((me + k) % N_DEV,),
                            device_id_type=pl.DeviceIdType.MESH)
    pl.semaphore_wait(barrier, N_DEV - 1)

    c_ref[pl.ds(me, 1)] = cnt_ref[...]

    rdmas = []
    for k in range(1, N_DEV):
        peer = (me + k) % N_DEV
        r = pltpu.make_async_remote_copy(
            src_ref=cnt_ref,
            dst_ref=c_ref.at[pl.ds(me, 1)],
            send_sem=send_sems.at[k - 1],
            recv_sem=recv_sems.at[me],
            device_id=(peer,),
            device_id_type=pl.DeviceIdType.MESH,
        )
        r.start()
        rdmas.append(r)

    for s in range(N_DEV):
        @pl.when(s != me)
        def _(s=s):
            rw = pltpu.make_async_remote_copy(
                src_ref=cnt_ref,
                dst_ref=c_ref.at[pl.ds(s, 1)],
                send_sem=send_sems.at[0],
                recv_sem=recv_sems.at[s],
                device_id=(s,),
                device_id_type=pl.DeviceIdType.MESH,
            )
            rw.wait_recv()

    for r in rdmas:
        r.wait_send()


def _scatter_body(cnt_ref, lo_ref, roff_ref, rcnt_ref, xs_ref, out_ref,
                  send_sems, recv_sems, local_sems):
    me = lax.axis_index("i")

    barrier = pltpu.get_barrier_semaphore()
    for k in range(1, N_DEV):
        pl.semaphore_signal(barrier, inc=1, device_id=((me + k) % N_DEV,),
                            device_id_type=pl.DeviceIdType.MESH)
    pl.semaphore_wait(barrier, N_DEV - 1)

    for d in range(N_DEV):
        c = cnt_ref[d]
        cur_s = lo_ref[d]
        cur_r = roff_ref[d]
        remote = me != d
        for b in range(NBITS - 1, -1, -1):
            sz = 1 << b
            has = ((c >> b) & 1) == 1

            @pl.when(has & remote)
            def _(d=d, b=b, sz=sz, cur_s=cur_s, cur_r=cur_r):
                r = pltpu.make_async_remote_copy(
                    src_ref=xs_ref.at[pl.ds(cur_s, sz)],
                    dst_ref=out_ref.at[pl.ds(cur_r, sz)],
                    send_sem=send_sems.at[d, b],
                    recv_sem=recv_sems.at[me, b],
                    device_id=(d,),
                    device_id_type=pl.DeviceIdType.MESH,
                )
                r.start()

            @pl.when(has & jnp.logical_not(remote))
            def _(b=b, sz=sz, cur_s=cur_s, cur_r=cur_r):
                cp = pltpu.make_async_copy(
                    xs_ref.at[pl.ds(cur_s, sz)],
                    out_ref.at[pl.ds(cur_r, sz)],
                    local_sems.at[b],
                )
                cp.start()

            inc = jnp.where(has, sz, 0)
            cur_s = cur_s + inc
            cur_r = cur_r + inc

    for d in range(N_DEV):
        c = cnt_ref[d]
        for b in range(NBITS - 1, -1, -1):
            sz = 1 << b
            has = ((c >> b) & 1) == 1

            @pl.when(has & (me == d))
            def _(b=b, sz=sz):
                cp = pltpu.make_async_copy(
                    xs_ref.at[pl.ds(0, sz)],
                    out_ref.at[pl.ds(0, sz)],
                    local_sems.at[b],
                )
                cp.wait()

    for d in range(N_DEV):
        c = cnt_ref[d]
        for b in range(NBITS - 1, -1, -1):
            sz = 1 << b
            has = ((c >> b) & 1) == 1

            @pl.when(has & (me != d))
            def _(d=d, b=b, sz=sz):
                r = pltpu.make_async_remote_copy(
                    src_ref=xs_ref.at[pl.ds(0, sz)],
                    dst_ref=out_ref.at[pl.ds(0, sz)],
                    send_sem=send_sems.at[d, b],
                    recv_sem=recv_sems.at[d, b],
                    device_id=(d,),
                    device_id_type=pl.DeviceIdType.MESH,
                )
                r.wait_send()

    for s in range(N_DEV):
        rc = rcnt_ref[s]
        for b in range(NBITS - 1, -1, -1):
            sz = 1 << b
            has = ((rc >> b) & 1) == 1

            @pl.when(has & (me != s))
            def _(s=s, b=b, sz=sz):
                r = pltpu.make_async_remote_copy(
                    src_ref=xs_ref.at[pl.ds(0, sz)],
                    dst_ref=out_ref.at[pl.ds(0, sz)],
                    send_sem=send_sems.at[s, b],
                    recv_sem=recv_sems.at[s, b],
                    device_id=(s,),
                    device_id_type=pl.DeviceIdType.MESH,
                )
                r.wait_recv()


def kernel(x, dest):
    me = lax.axis_index("i")
    d32 = dest.astype(jnp.int32)

    lorder = jnp.argsort(d32, stable=True)
    xs = jnp.take(x.astype(jnp.bfloat16), lorder, axis=0)
    cnt = jnp.bincount(d32, length=N_DEV).astype(jnp.int32)
    lo = (jnp.cumsum(cnt) - cnt).astype(jnp.int32)

    C = pl.pallas_call(
        _counts_body,
        out_shape=jax.ShapeDtypeStruct((N_DEV, 1, N_DEV), jnp.int32),
        in_specs=[pl.BlockSpec(memory_space=pltpu.VMEM)],
        out_specs=pl.BlockSpec(memory_space=pltpu.VMEM),
        scratch_shapes=[
            pltpu.SemaphoreType.DMA((N_DEV - 1,)),
            pltpu.SemaphoreType.DMA((N_DEV,)),
        ],
        compiler_params=pltpu.CompilerParams(collective_id=0),
    )(cnt.reshape(1, 1, N_DEV))
    C = C.reshape(N_DEV, N_DEV)

    rows_lt_me = jnp.arange(N_DEV)[:, None] < me
    roff = jnp.sum(jnp.where(rows_lt_me, C, 0), axis=0).astype(jnp.int32)
    rcnt = lax.dynamic_index_in_dim(C, me, axis=1, keepdims=False).astype(
        jnp.int32)

    out = pl.pallas_call(
        _scatter_body,
        out_shape=jax.ShapeDtypeStruct((ROWS, 8, D // 8), jnp.bfloat16),
        in_specs=[
            pl.BlockSpec(memory_space=pltpu.SMEM),
            pl.BlockSpec(memory_space=pltpu.SMEM),
            pl.BlockSpec(memory_space=pltpu.SMEM),
            pl.BlockSpec(memory_space=pltpu.SMEM),
            pl.BlockSpec(memory_space=pltpu.VMEM),
        ],
        out_specs=pl.BlockSpec(memory_space=pltpu.VMEM),
        scratch_shapes=[
            pltpu.SemaphoreType.DMA((N_DEV, NBITS)),
            pltpu.SemaphoreType.DMA((N_DEV, NBITS)),
            pltpu.SemaphoreType.DMA((NBITS,)),
        ],
        compiler_params=pltpu.CompilerParams(collective_id=1),
    )(cnt, lo, roff, rcnt, xs.reshape(ROWS, 8, D // 8))
    return out.reshape(ROWS, D)


# device time: 135515 ns/iter; 12.2476x vs baseline; 1.1564x over previous
import jax
import jax.numpy as jnp
from jax import lax
from jax.experimental import pallas as pl
from jax.experimental.pallas import tpu as pltpu

N_DEV = 16
ROWS = 4096
D = 1024
NBITS = 13


def _counts_body(cnt_ref, c_ref, send_sems, recv_sems):
    me = lax.axis_index("i")

    barrier = pltpu.get_barrier_semaphore()
    for k in range(1, N_DEV):
        pl.semaphore_signal(barrier, inc=1, device_id=((me + k) % N_DEV,),
                            device_id_type=pl.DeviceIdType.MESH)
    pl.semaphore_wait(barrier, N_DEV - 1)

    c_ref[pl.ds(me, 1)] = cnt_ref[...]

    rdmas = []
    for k in range(1, N_DEV):
        peer = (me + k) % N_DEV
        r = pltpu.make_async_remote_copy(
            src_ref=cnt_ref,
            dst_ref=c_ref.at[pl.ds(me, 1)],
            send_sem=send_sems.at[k - 1],
            recv_sem=recv_sems.at[me],
            device_id=(peer,),
            device_id_type=pl.DeviceIdType.MESH,
        )
        r.start()
        rdmas.append(r)

    for s in range(N_DEV):
        @pl.when(s != me)
        def _(s=s):
            rw = pltpu.make_async_remote_copy(
                src_ref=cnt_ref,
                dst_ref=c_ref.at[pl.ds(s, 1)],
                send_sem=send_sems.at[0],
                recv_sem=recv_sems.at[s],
                device_id=(s,),
                device_id_type=pl.DeviceIdType.MESH,
            )
            rw.wait_recv()

    for r in rdmas:
        r.wait_send()


def _scatter_body(cnt_ref, lo_ref, roff_ref, rcnt_ref, lorder_ref, x_ref,
                  out_ref, xs_ref, send_sems, recv_sems):
    me = lax.axis_index("i")

    barrier = pltpu.get_barrier_semaphore()
    for k in range(1, N_DEV):
        pl.semaphore_signal(barrier, inc=1, device_id=((me + k) % N_DEV,),
                            device_id_type=pl.DeviceIdType.MESH)
    pl.semaphore_wait(barrier, N_DEV - 1)

    for t in range(N_DEV - 1):
        d = (me + 1 + t) % N_DEV
        c = cnt_ref[d]
        base_s = lo_ref[d]
        base_r = roff_ref[d]

        def group_row(k, _, base_s=base_s):
            idx = lorder_ref[base_s + k]
            xs_ref[pl.ds(base_s + k, 1)] = x_ref[pl.ds(idx, 1)]
            return 0

        lax.fori_loop(0, c, group_row, 0)

        cur_s = base_s
        cur_r = base_r
        for b in range(NBITS - 1, -1, -1):
            sz = 1 << b
            has = ((c >> b) & 1) == 1

            @pl.when(has)
            def _(d=d, b=b, sz=sz, cur_s=cur_s, cur_r=cur_r):
                r = pltpu.make_async_remote_copy(
                    src_ref=xs_ref.at[pl.ds(cur_s, sz)],
                    dst_ref=out_ref.at[pl.ds(cur_r, sz)],
                    send_sem=send_sems.at[d, b],
                    recv_sem=recv_sems.at[me, b],
                    device_id=(d,),
                    device_id_type=pl.DeviceIdType.MESH,
                )
                r.start()

            inc = jnp.where(has, sz, 0)
            cur_s = cur_s + inc
            cur_r = cur_r + inc

    c_own = cnt_ref[me]
    base_own_s = lo_ref[me]
    base_own_r = roff_ref[me]

    def own_row(k, _):
        idx = lorder_ref[base_own_s + k]
        out_ref[pl.ds(base_own_r + k, 1)] = x_ref[pl.ds(idx, 1)]
        return 0

    lax.fori_loop(0, c_own, own_row, 0)

    for t in range(N_DEV - 1):
        d = (me + 1 + t) % N_DEV
        c = cnt_ref[d]
        for b in range(NBITS - 1, -1, -1):
            sz = 1 << b
            has = ((c >> b) & 1) == 1

            @pl.when(has)
            def _(d=d, b=b, sz=sz):
                r = pltpu.make_async_remote_copy(
                    src_ref=xs_ref.at[pl.ds(0, sz)],
                    dst_ref=out_ref.at[pl.ds(0, sz)],
                    send_sem=send_sems.at[d, b],
                    recv_sem=recv_sems.at[d, b],
                    device_id=(d,),
                    device_id_type=pl.DeviceIdType.MESH,
                )
                r.wait_send()

    for s in range(N_DEV):
        rc = rcnt_ref[s]
        for b in range(NBITS - 1, -1, -1):
            sz = 1 << b
            has = ((rc >> b) & 1) == 1

            @pl.when(has & (me != s))
            def _(s=s, b=b, sz=sz):
                r = pltpu.make_async_remote_copy(
                    src_ref=xs_ref.at[pl.ds(0, sz)],
                    dst_ref=out_ref.at[pl.ds(0, sz)],
                    send_sem=send_sems.at[s, b],
                    recv_sem=recv_sems.at[s, b],
                    device_id=(s,),
                    device_id_type=pl.DeviceIdType.MESH,
                )
                r.wait_recv()


def kernel(x, dest):
    me = lax.axis_index("i")
    d32 = dest.astype(jnp.int32)

    lorder = jnp.argsort(d32, stable=True).astype(jnp.int32)
    cnt = jnp.bincount(d32, length=N_DEV).astype(jnp.int32)
    lo = (jnp.cumsum(cnt) - cnt).astype(jnp.int32)

    C = pl.pallas_call(
        _counts_body,
        out_shape=jax.ShapeDtypeStruct((N_DEV, 1, N_DEV), jnp.int32),
        in_specs=[pl.BlockSpec(memory_space=pltpu.VMEM)],
        out_specs=pl.BlockSpec(memory_space=pltpu.VMEM),
        scratch_shapes=[
            pltpu.SemaphoreType.DMA((N_DEV - 1,)),
            pltpu.SemaphoreType.DMA((N_DEV,)),
        ],
        compiler_params=pltpu.CompilerParams(collective_id=0),
    )(cnt.reshape(1, 1, N_DEV))
    C = C.reshape(N_DEV, N_DEV)

    rows_lt_me = jnp.arange(N_DEV)[:, None] < me
    roff = jnp.sum(jnp.where(rows_lt_me, C, 0), axis=0).astype(jnp.int32)
    rcnt = lax.dynamic_index_in_dim(C, me, axis=1, keepdims=False).astype(
        jnp.int32)

    x3 = x.astype(jnp.bfloat16).reshape(ROWS, 8, D // 8)
    out = pl.pallas_call(
        _scatter_body,
        out_shape=jax.ShapeDtypeStruct((ROWS, 8, D // 8), jnp.bfloat16),
        in_specs=[
            pl.BlockSpec(memory_space=pltpu.SMEM),
            pl.BlockSpec(memory_space=pltpu.SMEM),
            pl.BlockSpec(memory_space=pltpu.SMEM),
            pl.BlockSpec(memory_space=pltpu.SMEM),
            pl.BlockSpec(memory_space=pltpu.SMEM),
            pl.BlockSpec(memory_space=pltpu.VMEM),
        ],
        out_specs=pl.BlockSpec(memory_space=pltpu.VMEM),
        scratch_shapes=[
            pltpu.VMEM((ROWS, 8, D // 8), jnp.bfloat16),
            pltpu.SemaphoreType.DMA((N_DEV, NBITS)),
            pltpu.SemaphoreType.DMA((N_DEV, NBITS)),
        ],
        compiler_params=pltpu.CompilerParams(collective_id=1),
    )(cnt, lo, roff, rcnt, lorder, x3)
    return out.reshape(ROWS, D)


# device time: 132424 ns/iter; 12.5335x vs baseline; 1.0233x over previous
import jax
import jax.numpy as jnp
from jax import lax
from jax.experimental import pallas as pl
from jax.experimental.pallas import tpu as pltpu

N_DEV = 16
ROWS = 4096
D = 1024
NBITS = 13


def _a2av_body(cnt_ref, lo_ref, cntv_ref, lorder_ref, x_ref,
               out_ref, xs_ref, c_ref, c_smem, roff_ref,
               csend_sems, crecv_sems, copy_sem, send_sems, recv_sems):
    me = lax.axis_index("i")

    barrier = pltpu.get_barrier_semaphore()
    for k in range(1, N_DEV):
        pl.semaphore_signal(barrier, inc=1, device_id=((me + k) % N_DEV,),
                            device_id_type=pl.DeviceIdType.MESH)
    pl.semaphore_wait(barrier, N_DEV - 1)

    c_ref[pl.ds(me, 1)] = cntv_ref[...]
    count_rdmas = []
    for k in range(1, N_DEV):
        peer = (me + k) % N_DEV
        r = pltpu.make_async_remote_copy(
            src_ref=cntv_ref,
            dst_ref=c_ref.at[pl.ds(me, 1)],
            send_sem=csend_sems.at[k - 1],
            recv_sem=crecv_sems.at[me],
            device_id=(peer,),
            device_id_type=pl.DeviceIdType.MESH,
        )
        r.start()
        count_rdmas.append(r)

    for s in range(N_DEV):
        @pl.when(s != me)
        def _(s=s):
            rw = pltpu.make_async_remote_copy(
                src_ref=cntv_ref,
                dst_ref=c_ref.at[pl.ds(s, 1)],
                send_sem=csend_sems.at[0],
                recv_sem=crecv_sems.at[s],
                device_id=(s,),
                device_id_type=pl.DeviceIdType.MESH,
            )
            rw.wait_recv()
    for r in count_rdmas:
        r.wait_send()

    cp = pltpu.make_async_copy(c_ref, c_smem, copy_sem)
    cp.start()
    cp.wait()

    for d in range(N_DEV):
        acc = jnp.int32(0)
        for s in range(N_DEV):
            acc = acc + jnp.where(s < me, c_smem[s, 0, d], 0)
        roff_ref[d] = acc

    for t in range(N_DEV - 1):
        d = (me + 1 + t) % N_DEV
        c = cnt_ref[d]
        base_s = lo_ref[d]
        base_r = roff_ref[d]

        def group_row(k, _, base_s=base_s):
            idx = lorder_ref[base_s + k]
            xs_ref[pl.ds(base_s + k, 1)] = x_ref[pl.ds(idx, 1)]
            return 0

        lax.fori_loop(0, c, group_row, 0)

        cur_s = base_s
        cur_r = base_r
        for b in range(NBITS - 1, -1, -1):
            sz = 1 << b
            has = ((c >> b) & 1) == 1

            @pl.when(has)
            def _(d=d, b=b, sz=sz, cur_s=cur_s, cur_r=cur_r):
                r = pltpu.make_async_remote_copy(
                    src_ref=xs_ref.at[pl.ds(cur_s, sz)],
                    dst_ref=out_ref.at[pl.ds(cur_r, sz)],
                    send_sem=send_sems.at[d, b],
                    recv_sem=recv_sems.at[me, b],
                    device_id=(d,),
                    device_id_type=pl.DeviceIdType.MESH,
                )
                r.start()

            inc = jnp.where(has, sz, 0)
            cur_s = cur_s + inc
            cur_r = cur_r + inc

    base_own_s = lo_ref[me]
    base_own_r = roff_ref[me]

    def own_row(k, _):
        idx = lorder_ref[base_own_s + k]
        out_ref[pl.ds(base_own_r + k, 1)] = x_ref[pl.ds(idx, 1)]
        return 0

    lax.fori_loop(0, cnt_ref[me], own_row, 0)

    for t in range(N_DEV - 1):
        d = (me + 1 + t) % N_DEV
        c = cnt_ref[d]
        for b in range(NBITS - 1, -1, -1):
            sz = 1 << b
            has = ((c >> b) & 1) == 1

            @pl.when(has)
            def _(d=d, b=b, sz=sz):
                r = pltpu.make_async_remote_copy(
                    src_ref=xs_ref.at[pl.ds(0, sz)],
                    dst_ref=out_ref.at[pl.ds(0, sz)],
                    send_sem=send_sems.at[d, b],
                    recv_sem=recv_sems.at[d, b],
                    device_id=(d,),
                    device_id_type=pl.DeviceIdType.MESH,
                )
                r.wait_send()

    for s in range(N_DEV):
        rc = c_smem[s, 0, me]
        for b in range(NBITS - 1, -1, -1):
            sz = 1 << b
            has = ((rc >> b) & 1) == 1

            @pl.when(has & (me != s))
            def _(s=s, b=b, sz=sz):
                r = pltpu.make_async_remote_copy(
                    src_ref=xs_ref.at[pl.ds(0, sz)],
                    dst_ref=out_ref.at[pl.ds(0, sz)],
                    send_sem=send_sems.at[s, b],
                    recv_sem=recv_sems.at[s, b],
                    device_id=(s,),
                    device_id_type=pl.DeviceIdType.MESH,
                )
                r.wait_recv()


def kernel(x, dest):
    d32 = dest.astype(jnp.int32)

    lorder = jnp.argsort(d32, stable=True).astype(jnp.int32)
    cnt = jnp.bincount(d32, length=N_DEV).astype(jnp.int32)
    lo = (jnp.cumsum(cnt) - cnt).astype(jnp.int32)

    x3 = x.astype(jnp.bfloat16).reshape(ROWS, 8, D // 8)
    out = pl.pallas_call(
        _a2av_body,
        out_shape=jax.ShapeDtypeStruct((ROWS, 8, D // 8), jnp.bfloat16),
        in_specs=[
            pl.BlockSpec(memory_space=pltpu.SMEM),
            pl.BlockSpec(memory_space=pltpu.SMEM),
            pl.BlockSpec(memory_space=pltpu.VMEM),
            pl.BlockSpec(memory_space=pltpu.SMEM),
            pl.BlockSpec(memory_space=pltpu.VMEM),
        ],
        out_specs=pl.BlockSpec(memory_space=pltpu.VMEM),
        scratch_shapes=[
            pltpu.VMEM((ROWS, 8, D // 8), jnp.bfloat16),
            pltpu.VMEM((N_DEV, 1, N_DEV), jnp.int32),
            pltpu.SMEM((N_DEV, 1, N_DEV), jnp.int32),
            pltpu.SMEM((N_DEV,), jnp.int32),
            pltpu.SemaphoreType.DMA((N_DEV - 1,)),
            pltpu.SemaphoreType.DMA((N_DEV,)),
            pltpu.SemaphoreType.DMA,
            pltpu.SemaphoreType.DMA((N_DEV, NBITS)),
            pltpu.SemaphoreType.DMA((N_DEV, NBITS)),
        ],
        compiler_params=pltpu.CompilerParams(collective_id=0),
    )(cnt, lo, cnt.reshape(1, 1, N_DEV), lorder, x3)
    return out.reshape(ROWS, D)
